# Initial kernel scaffold; baseline (speedup 1.0000x reference)
import jax
import jax.numpy as jnp
from jax.experimental import pallas as pl


def kernel(x, edge_index, W_l, b_l, W_r):
    raise NotImplementedError("write your pallas kernel here")


# speedup vs baseline: 6.2617x; 6.2617x over previous
import functools

import jax
import jax.numpy as jnp
from jax import lax
from jax.experimental import pallas as pl
from jax.experimental.pallas import tpu as pltpu
from jax.experimental.pallas import tpu_sc as plsc

NCORES = 2
NSUB = 16
NTILES = NCORES * NSUB
CHUNK = 128
ZROWS = 64


def _round_up(a, b):
  return (a + b - 1) // b * b


def _sc_aggregate(x, src3, dst3):
  n_nodes, d = x.shape
  chunks_per_tile = src3.shape[1]
  acc_rows = _round_up(n_nodes + 1, NSUB * ZROWS)
  zdmas_per_tile = acc_rows // NSUB // ZROWS
  rows_per_tile = acc_rows // NSUB

  mesh = plsc.VectorSubcoreMesh(core_axis_name="c", subcore_axis_name="s")

  @functools.partial(
      pl.kernel,
      out_type=(jax.ShapeDtypeStruct((NCORES, acc_rows, d), jnp.float32),
                jax.ShapeDtypeStruct((NCORES, acc_rows), jnp.float32)),
      mesh=mesh,
      scratch_types=[
          pltpu.VMEM((chunks_per_tile, CHUNK), jnp.int32),
          pltpu.VMEM((chunks_per_tile, CHUNK), jnp.int32),
          pltpu.VMEM((CHUNK, d), jnp.float32),
          pltpu.VMEM((ZROWS, d), jnp.float32),
          pltpu.VMEM((rows_per_tile,), jnp.float32),
          pltpu.VMEM((CHUNK,), jnp.float32),
          pltpu.VMEM_SHARED((acc_rows, d), jnp.float32),
          pltpu.VMEM_SHARED((acc_rows,), jnp.float32),
          pltpu.SemaphoreType.DMA,
      ],
  )
  def k(x_hbm, src_hbm, dst_hbm, out_hbm, cnt_hbm, src_v, dst_v, rows_v,
        zbuf, zero1_v, ones_v, acc_sh, cnt_sh, sem):
    cid = lax.axis_index("c")
    sid = lax.axis_index("s")
    wid = sid * NCORES + cid

    def zrow(i, _):
      def zcol(kk, _):
        zbuf[i, pl.ds(kk * 16, 16)] = jnp.zeros((16,), jnp.float32)
        return 0
      return lax.fori_loop(0, d // 16, zcol, 0)
    lax.fori_loop(0, ZROWS, zrow, 0)

    def z1d(i, _):
      zero1_v[pl.ds(i * 16, 16)] = jnp.zeros((16,), jnp.float32)
      return 0
    lax.fori_loop(0, rows_per_tile // 16, z1d, 0)

    def o1d(i, _):
      ones_v[pl.ds(i * 16, 16)] = jnp.ones((16,), jnp.float32)
      return 0
    lax.fori_loop(0, CHUNK // 16, o1d, 0)

    def zdma(b, _):
      base = sid * rows_per_tile + b * ZROWS
      pltpu.sync_copy(zbuf, acc_sh.at[pl.ds(base, ZROWS)])
      return 0
    lax.fori_loop(0, zdmas_per_tile, zdma, 0)
    pltpu.sync_copy(zero1_v, cnt_sh.at[pl.ds(sid * rows_per_tile,
                                             rows_per_tile)])

    pltpu.sync_copy(src_hbm.at[wid], src_v)
    pltpu.sync_copy(dst_hbm.at[wid], dst_v)

    plsc.subcore_barrier()

    def chunk_body(j, _):
      pltpu.async_copy(x_hbm.at[src_v.at[j]], rows_v, sem).wait()
      pltpu.sync_copy(rows_v, acc_sh.at[dst_v.at[j]], add=True)
      pltpu.sync_copy(ones_v, cnt_sh.at[dst_v.at[j]], add=True)
      return 0
    lax.fori_loop(0, chunks_per_tile, chunk_body, 0)

    plsc.subcore_barrier()

    pltpu.sync_copy(
        acc_sh.at[pl.ds(sid * rows_per_tile, rows_per_tile)],
        out_hbm.at[cid, pl.ds(sid * rows_per_tile, rows_per_tile)])
    pltpu.sync_copy(
        cnt_sh.at[pl.ds(sid * rows_per_tile, rows_per_tile)],
        cnt_hbm.at[cid, pl.ds(sid * rows_per_tile, rows_per_tile)])

  return k(x, src3, dst3)


def _tc_body(part_ref, cnt_ref, x_ref, wl_ref, bl_ref, wr_ref, o_ref):
  s = part_ref[0] + part_ref[1]
  c = cnt_ref[0] + cnt_ref[1]
  mean = s / jnp.maximum(c, 1.0)
  z = (lax.dot_general(mean, wl_ref[...], (((1,), (1,)), ((), ())),
                       preferred_element_type=jnp.float32)
       + bl_ref[...]
       + lax.dot_general(x_ref[...], wr_ref[...], (((1,), (1,)), ((), ())),
                         preferred_element_type=jnp.float32))
  m = jnp.max(z, axis=1, keepdims=True)
  e = z - m
  lse = jnp.log(jnp.sum(jnp.exp(e), axis=1, keepdims=True))
  o_ref[...] = e - lse


def _tc_finish(part, cnt, x, w_l, b_l, w_r):
  n, d_in = x.shape
  d_out = w_l.shape[0]
  blk = 400
  return pl.pallas_call(
      _tc_body,
      grid=(n // blk,),
      in_specs=[
          pl.BlockSpec((NCORES, blk, d_in), lambda i: (0, i, 0)),
          pl.BlockSpec((NCORES, blk, 1), lambda i: (0, i, 0)),
          pl.BlockSpec((blk, d_in), lambda i: (i, 0)),
          pl.BlockSpec((d_out, d_in), lambda i: (0, 0)),
          pl.BlockSpec((1, d_out), lambda i: (0, 0)),
          pl.BlockSpec((d_out, d_in), lambda i: (0, 0)),
      ],
      out_specs=pl.BlockSpec((blk, d_out), lambda i: (i, 0)),
      out_shape=jax.ShapeDtypeStruct((n, d_out), jnp.float32),
  )(part, cnt, x, w_l, b_l.reshape(1, d_out), w_r)


@jax.jit
def kernel(x, edge_index, W_l, b_l, W_r):
  n, d_in = x.shape
  e = edge_index.shape[1]
  chunks_per_tile = -(-e // (NTILES * CHUNK))
  e_pad = NTILES * chunks_per_tile * CHUNK
  src = edge_index[0].astype(jnp.int32)
  dst = edge_index[1].astype(jnp.int32)
  src = jnp.concatenate([src, jnp.zeros((e_pad - e,), jnp.int32)])
  dst = jnp.concatenate([dst, jnp.full((e_pad - e,), n, jnp.int32)])
  src3 = src.reshape(NTILES, chunks_per_tile, CHUNK)
  dst3 = dst.reshape(NTILES, chunks_per_tile, CHUNK)
  part, cnt = _sc_aggregate(x, src3, dst3)
  return _tc_finish(part, cnt[:, :, None], x, W_l, b_l, W_r)
